# baseline (speedup 1.0000x reference)
import functools

import jax
import jax.numpy as jnp
from jax import lax
from jax.experimental import pallas as pl
from jax.experimental.pallas import tpu as pltpu
from jax.experimental.pallas import tpu_sc as plsc

NC, NS, L = 2, 16, 16


def _deg_body(dst_hbm, degp_hbm, dstv, hist, nrows, ept):
    c = lax.axis_index("c")
    s = lax.axis_index("s")
    w = c * NS + s
    pltpu.sync_copy(dst_hbm.at[w], dstv)

    def zero(i, carry):
        hist[pl.ds(i * L, L)] = jnp.zeros((L,), jnp.float32)
        return carry

    lax.fori_loop(0, nrows // L, zero, 0)

    ones = jnp.ones((L,), jnp.float32)

    def acc(i, carry):
        idx = dstv[pl.ds(i * L, L)]
        plsc.addupdate_scatter(hist, [idx], ones)
        return carry

    lax.fori_loop(0, ept // L, acc, 0)
    pltpu.sync_copy(hist, degp_hbm.at[w])


def _dinv_body(degp_ref, dinv_ref):
    deg = jnp.sum(degp_ref[...], axis=0) + 1.0
    dinv_ref[...] = lax.rsqrt(deg)[:, None]


def _xw_body(x_ref, w_ref, dinv_ref, y_ref):
    xw = jnp.dot(x_ref[0], w_ref[0], preferred_element_type=jnp.float32)
    y_ref[0, 0] = xw * dinv_ref[...]


def _agg_body(y_hbm, src_hbm, dst_hbm, S_hbm, src_v, dst_v, b0,
              zbuf, S_sh, g0,
              ngraphs, nchunks, ept, rows_per_tile, hh):
    c = lax.axis_index("c")
    s = lax.axis_index("s")
    pltpu.sync_copy(src_hbm.at[s], src_v)
    pltpu.sync_copy(dst_hbm.at[s], dst_v)

    def zero(i, carry):
        r = i // (hh // L)
        l = i % (hh // L)
        zbuf[r, pl.ds(l * L, L)] = jnp.zeros((L,), jnp.float32)
        return carry

    lax.fori_loop(0, 64 * hh // L, zero, 0)

    base = s * rows_per_tile
    for g in range(ngraphs):
        for k in range(rows_per_tile // 64):
            pltpu.sync_copy(zbuf, S_sh.at[pl.ds(base + k * 64, 64)])
        plsc.subcore_barrier()

        def chunk(q, carry):
            pltpu.async_copy(y_hbm.at[c].at[g].at[src_v.at[q]], b0,
                             g0).wait()
            pltpu.sync_copy(b0, S_sh.at[dst_v.at[q]], add=True)
            return carry

        lax.fori_loop(0, nchunks, chunk, 0)
        plsc.subcore_barrier()
        for k in range(rows_per_tile // 128):
            pltpu.sync_copy(S_sh.at[pl.ds(base + k * 128, 128)],
                            S_hbm.at[c, g, pl.ds(base + k * 128, 128)])
        plsc.subcore_barrier()


def _out_body(S0_ref, S1_ref, y0_ref, y1_ref, dinv_ref, b_ref, o_ref):
    S = jnp.concatenate([S0_ref[0, 0], S1_ref[0, 0]], axis=-1)
    y = jnp.concatenate([y0_ref[0, 0], y1_ref[0, 0]], axis=-1)
    o_ref[0] = dinv_ref[...] * (S + y) + b_ref[...]


def kernel(x, edge_index, W, b):
    B, T, N, F = x.shape
    H = W.shape[1]
    G = B * T
    E = edge_index.shape[1]
    HH = H // 2

    ept = ((E // NS + 511) // 512) * 512
    nchunks = ept // 128
    EP = NS * ept
    NR = ((N + (NS * 128) - 1) // (NS * 128)) * (NS * 128)
    rows_per_tile = NR // NS

    src = edge_index[0]
    dst = edge_index[1]
    pad = EP - E
    srcp = jnp.concatenate([src, jnp.zeros((pad,), jnp.int32)])
    dstp = jnp.concatenate([dst, jnp.full((pad,), N, jnp.int32)])
    srcT = srcp.reshape(NS, nchunks, 128)
    dstT = dstp.reshape(NS, nchunks, 128)
    dstA = dstp.reshape(NC * NS, EP // (NC * NS))

    mesh = plsc.VectorSubcoreMesh(core_axis_name="c", subcore_axis_name="s")

    deg_parts = pl.kernel(
        functools.partial(_deg_body, nrows=NR, ept=EP // (NC * NS)),
        out_type=jax.ShapeDtypeStruct((NC * NS, NR), jnp.float32),
        mesh=mesh,
        scratch_types=[
            pltpu.VMEM((EP // (NC * NS),), jnp.int32),
            pltpu.VMEM((NR,), jnp.float32),
        ],
        compiler_params=pltpu.CompilerParams(needs_layout_passes=False),
    )(dstA)

    dinv = pl.pallas_call(
        _dinv_body,
        out_shape=jax.ShapeDtypeStruct((NR, 1), jnp.float32),
    )(deg_parts)

    RB = 1000
    nrb = N // RB
    xg = x.reshape(G, N, F)
    Wh = jnp.stack([W[:, :HH], W[:, HH:]])
    y = pl.pallas_call(
        _xw_body,
        grid=(NC, G, nrb),
        in_specs=[
            pl.BlockSpec((1, RB, F), lambda h, g, i: (g, i, 0)),
            pl.BlockSpec((1, F, HH), lambda h, g, i: (h, 0, 0)),
            pl.BlockSpec((RB, 1), lambda h, g, i: (i, 0)),
        ],
        out_specs=pl.BlockSpec((1, 1, RB, HH), lambda h, g, i: (h, g, i, 0)),
        out_shape=jax.ShapeDtypeStruct((NC, G, N, HH), jnp.float32),
    )(xg, Wh, dinv)

    S = pl.kernel(
        functools.partial(_agg_body, ngraphs=G, nchunks=nchunks, ept=ept,
                          rows_per_tile=rows_per_tile, hh=HH),
        out_type=jax.ShapeDtypeStruct((NC, G, NR, HH), jnp.float32),
        mesh=mesh,
        scratch_types=[
            pltpu.VMEM((nchunks, 128), jnp.int32),
            pltpu.VMEM((nchunks, 128), jnp.int32),
            pltpu.VMEM((128, HH), jnp.float32),
            pltpu.VMEM((64, HH), jnp.float32),
            pltpu.VMEM_SHARED((NR, HH), jnp.float32),
            pltpu.SemaphoreType.DMA,
        ],
        compiler_params=pltpu.CompilerParams(needs_layout_passes=False,
                                            use_tc_tiling_on_sc=False),
    )(y, srcT, dstT)

    out = pl.pallas_call(
        _out_body,
        grid=(G, nrb),
        in_specs=[
            pl.BlockSpec((1, 1, RB, HH), lambda g, i: (0, g, i, 0)),
            pl.BlockSpec((1, 1, RB, HH), lambda g, i: (1, g, i, 0)),
            pl.BlockSpec((1, 1, RB, HH), lambda g, i: (0, g, i, 0)),
            pl.BlockSpec((1, 1, RB, HH), lambda g, i: (1, g, i, 0)),
            pl.BlockSpec((RB, 1), lambda g, i: (i, 0)),
            pl.BlockSpec((1, H), lambda g, i: (0, 0)),
        ],
        out_specs=pl.BlockSpec((1, RB, H), lambda g, i: (g, i, 0)),
        out_shape=jax.ShapeDtypeStruct((G, N, H), jnp.float32),
    )(S, S, y, y, dinv, b.reshape(1, H))

    return out.reshape(B, T, N, H)


# speedup vs baseline: 1.5306x; 1.5306x over previous
import functools

import jax
import jax.numpy as jnp
from jax import lax
from jax.experimental import pallas as pl
from jax.experimental.pallas import tpu as pltpu
from jax.experimental.pallas import tpu_sc as plsc

NC, NS, L = 2, 16, 16


def _deg_body(dst_hbm, degp_hbm, dstv, hist, nrows, ept):
    c = lax.axis_index("c")
    s = lax.axis_index("s")
    w = c * NS + s
    pltpu.sync_copy(dst_hbm.at[w], dstv)

    def zero(i, carry):
        hist[pl.ds(i * L, L)] = jnp.zeros((L,), jnp.float32)
        return carry

    lax.fori_loop(0, nrows // L, zero, 0)

    ones = jnp.ones((L,), jnp.float32)

    def acc(i, carry):
        idx = dstv[pl.ds(i * L, L)]
        plsc.addupdate_scatter(hist, [idx], ones)
        return carry

    lax.fori_loop(0, ept // L, acc, 0)
    pltpu.sync_copy(hist, degp_hbm.at[w])


def _dinv_body(degp_ref, dinv_ref):
    deg = jnp.sum(degp_ref[...], axis=0) + 1.0
    dinv_ref[...] = lax.rsqrt(deg)[:, None]


def _xw_body(x_ref, w_ref, dinv_ref, y_ref):
    xw = jnp.dot(x_ref[0], w_ref[0], preferred_element_type=jnp.float32)
    y_ref[0, 0] = xw * dinv_ref[...]


def _agg_body(y_hbm, src_hbm, dst_hbm, S_hbm, src_v, dst_v, b0,
              zbuf, S_sh, g0,
              ngraphs, nchunks, ept, rows_per_tile, hh):
    c = lax.axis_index("c")
    s = lax.axis_index("s")
    pltpu.sync_copy(src_hbm.at[s], src_v)
    pltpu.sync_copy(dst_hbm.at[s], dst_v)

    def zero(i, carry):
        r = i // (hh // L)
        l = i % (hh // L)
        zbuf[r, pl.ds(l * L, L)] = jnp.zeros((L,), jnp.float32)
        return carry

    lax.fori_loop(0, 128 * hh // L, zero, 0)

    base = s * rows_per_tile
    for g in range(ngraphs):
        for k in range(rows_per_tile // 128):
            pltpu.sync_copy(zbuf, S_sh.at[pl.ds(base + k * 128, 128)])
        plsc.subcore_barrier()

        def chunk(q, carry):
            pltpu.async_copy(y_hbm.at[c].at[g].at[src_v.at[q]], b0,
                             g0).wait()
            pltpu.sync_copy(b0, S_sh.at[dst_v.at[q]], add=True)
            return carry

        lax.fori_loop(0, nchunks, chunk, 0)
        plsc.subcore_barrier()
        for k in range(rows_per_tile // 128):
            pltpu.sync_copy(S_sh.at[pl.ds(base + k * 128, 128)],
                            S_hbm.at[c, g, pl.ds(base + k * 128, 128)])
        plsc.subcore_barrier()


def _out_body(S0_ref, S1_ref, y0_ref, y1_ref, dinv_ref, b_ref, o_ref):
    S = jnp.concatenate([S0_ref[0, 0], S1_ref[0, 0]], axis=-1)
    y = jnp.concatenate([y0_ref[0, 0], y1_ref[0, 0]], axis=-1)
    o_ref[0] = dinv_ref[...] * (S + y) + b_ref[...]


def kernel(x, edge_index, W, b):
    B, T, N, F = x.shape
    H = W.shape[1]
    G = B * T
    E = edge_index.shape[1]
    HH = H // 2

    ept = ((E // NS + 127) // 128) * 128
    nchunks = ept // 128
    EP = NS * ept
    NR = ((N + (NS * 128) - 1) // (NS * 128)) * (NS * 128)
    rows_per_tile = NR // NS

    src = edge_index[0]
    dst = edge_index[1]
    pad = EP - E
    srcp = jnp.concatenate([src, jnp.zeros((pad,), jnp.int32)])
    dstp = jnp.concatenate([dst, jnp.full((pad,), N, jnp.int32)])
    srcT = srcp.reshape(NS, nchunks, 128)
    dstT = dstp.reshape(NS, nchunks, 128)
    dstA = dstp.reshape(NC * NS, EP // (NC * NS))

    mesh = plsc.VectorSubcoreMesh(core_axis_name="c", subcore_axis_name="s")

    deg_parts = pl.kernel(
        functools.partial(_deg_body, nrows=NR, ept=EP // (NC * NS)),
        out_type=jax.ShapeDtypeStruct((NC * NS, NR), jnp.float32),
        mesh=mesh,
        scratch_types=[
            pltpu.VMEM((EP // (NC * NS),), jnp.int32),
            pltpu.VMEM((NR,), jnp.float32),
        ],
        compiler_params=pltpu.CompilerParams(needs_layout_passes=False),
    )(dstA)

    dinv = pl.pallas_call(
        _dinv_body,
        out_shape=jax.ShapeDtypeStruct((NR, 1), jnp.float32),
    )(deg_parts)

    RB = 1000
    nrb = N // RB
    xg = x.reshape(G, N, F)
    Wh = jnp.stack([W[:, :HH], W[:, HH:]])
    y = pl.pallas_call(
        _xw_body,
        grid=(NC, G, nrb),
        in_specs=[
            pl.BlockSpec((1, RB, F), lambda h, g, i: (g, i, 0)),
            pl.BlockSpec((1, F, HH), lambda h, g, i: (h, 0, 0)),
            pl.BlockSpec((RB, 1), lambda h, g, i: (i, 0)),
        ],
        out_specs=pl.BlockSpec((1, 1, RB, HH), lambda h, g, i: (h, g, i, 0)),
        out_shape=jax.ShapeDtypeStruct((NC, G, N, HH), jnp.float32),
    )(xg, Wh, dinv)

    S = pl.kernel(
        functools.partial(_agg_body, ngraphs=G, nchunks=nchunks, ept=ept,
                          rows_per_tile=rows_per_tile, hh=HH),
        out_type=jax.ShapeDtypeStruct((NC, G, NR, HH), jnp.float32),
        mesh=mesh,
        scratch_types=[
            pltpu.VMEM((nchunks, 128), jnp.int32),
            pltpu.VMEM((nchunks, 128), jnp.int32),
            pltpu.VMEM((128, HH), jnp.float32),
            pltpu.VMEM((128, HH), jnp.float32),
            pltpu.VMEM_SHARED((NR, HH), jnp.float32),
            pltpu.SemaphoreType.DMA,
        ],
        compiler_params=pltpu.CompilerParams(needs_layout_passes=False,
                                            use_tc_tiling_on_sc=False),
    )(y, srcT, dstT)

    out = pl.pallas_call(
        _out_body,
        grid=(G, nrb),
        in_specs=[
            pl.BlockSpec((1, 1, RB, HH), lambda g, i: (0, g, i, 0)),
            pl.BlockSpec((1, 1, RB, HH), lambda g, i: (1, g, i, 0)),
            pl.BlockSpec((1, 1, RB, HH), lambda g, i: (0, g, i, 0)),
            pl.BlockSpec((1, 1, RB, HH), lambda g, i: (1, g, i, 0)),
            pl.BlockSpec((RB, 1), lambda g, i: (i, 0)),
            pl.BlockSpec((1, H), lambda g, i: (0, 0)),
        ],
        out_specs=pl.BlockSpec((1, RB, H), lambda g, i: (g, i, 0)),
        out_shape=jax.ShapeDtypeStruct((G, N, H), jnp.float32),
    )(S, S, y, y, dinv, b.reshape(1, H))

    return out.reshape(B, T, N, H)


# speedup vs baseline: 1.7770x; 1.1610x over previous
import functools

import jax
import jax.numpy as jnp
from jax import lax
from jax.experimental import pallas as pl
from jax.experimental.pallas import tpu as pltpu
from jax.experimental.pallas import tpu_sc as plsc

NC, NS, L = 2, 16, 16


def _deg_body(dst_hbm, degp_hbm, dstv, hist, nrows, ept):
    c = lax.axis_index("c")
    s = lax.axis_index("s")
    w = c * NS + s
    pltpu.sync_copy(dst_hbm.at[w], dstv)

    def zero(i, carry):
        hist[pl.ds(i * L, L)] = jnp.zeros((L,), jnp.float32)
        return carry

    lax.fori_loop(0, nrows // L, zero, 0)

    ones = jnp.ones((L,), jnp.float32)

    def acc(i, carry):
        idx = dstv[pl.ds(i * L, L)]
        plsc.addupdate_scatter(hist, [idx], ones)
        return carry

    lax.fori_loop(0, ept // L, acc, 0)
    pltpu.sync_copy(hist, degp_hbm.at[w])


def _dinv_body(degp_ref, dinv_ref):
    deg = jnp.sum(degp_ref[...], axis=0) + 1.0
    dinv_ref[...] = lax.rsqrt(deg)[:, None]


def _xw_body(x_ref, w_ref, dinv_ref, y_ref):
    xw = jnp.dot(x_ref[0], w_ref[0], preferred_element_type=jnp.float32)
    y_ref[0, 0] = xw * dinv_ref[...]


def _agg_body(y_hbm, src_hbm, dst_hbm, S_hbm, src_v, dst_v, b0,
              S_sh, tab_sh, g0,
              ngraphs, nchunks, ept, rows_per_tile, hh, nnodes):
    c = lax.axis_index("c")
    s = lax.axis_index("s")
    pltpu.sync_copy(src_hbm.at[s], src_v)
    pltpu.sync_copy(dst_hbm.at[s], dst_v)

    def zero(i, carry):
        r = i // (hh // L)
        l = i % (hh // L)
        b0[r, pl.ds(l * L, L)] = jnp.zeros((L,), jnp.float32)
        return carry

    base = s * rows_per_tile
    rpt = nnodes // NS
    for g in range(ngraphs):
        lax.fori_loop(0, 128 * hh // L, zero, 0)
        for k in range(rows_per_tile // 128):
            pltpu.sync_copy(b0, S_sh.at[pl.ds(base + k * 128, 128)])
        pltpu.sync_copy(y_hbm.at[c, g, pl.ds(s * rpt, rpt)],
                        tab_sh.at[pl.ds(s * rpt, rpt)])
        plsc.subcore_barrier()

        def chunk(q, carry):
            pltpu.async_copy(tab_sh.at[src_v.at[q]], b0, g0).wait()
            pltpu.sync_copy(b0, S_sh.at[dst_v.at[q]], add=True)
            return carry

        lax.fori_loop(0, nchunks, chunk, 0)
        plsc.subcore_barrier()
        for k in range(rows_per_tile // 128):
            pltpu.sync_copy(S_sh.at[pl.ds(base + k * 128, 128)],
                            S_hbm.at[c, g, pl.ds(base + k * 128, 128)])
        plsc.subcore_barrier()


def _out_body(S0_ref, S1_ref, y0_ref, y1_ref, dinv_ref, b_ref, o_ref):
    S = jnp.concatenate([S0_ref[0, 0], S1_ref[0, 0]], axis=-1)
    y = jnp.concatenate([y0_ref[0, 0], y1_ref[0, 0]], axis=-1)
    o_ref[0] = dinv_ref[...] * (S + y) + b_ref[...]


def kernel(x, edge_index, W, b):
    B, T, N, F = x.shape
    H = W.shape[1]
    G = B * T
    E = edge_index.shape[1]
    HH = H // 2

    ept = ((E // NS + 127) // 128) * 128
    nchunks = ept // 128
    EP = NS * ept
    NR = ((N + (NS * 128) - 1) // (NS * 128)) * (NS * 128)
    rows_per_tile = NR // NS

    src = edge_index[0]
    dst = edge_index[1]
    pad = EP - E
    srcp = jnp.concatenate([src, jnp.zeros((pad,), jnp.int32)])
    dstp = jnp.concatenate([dst, jnp.full((pad,), N, jnp.int32)])
    srcT = srcp.reshape(NS, nchunks, 128)
    dstT = dstp.reshape(NS, nchunks, 128)
    dstA = dstp.reshape(NC * NS, EP // (NC * NS))

    mesh = plsc.VectorSubcoreMesh(core_axis_name="c", subcore_axis_name="s")

    deg_parts = pl.kernel(
        functools.partial(_deg_body, nrows=NR, ept=EP // (NC * NS)),
        out_type=jax.ShapeDtypeStruct((NC * NS, NR), jnp.float32),
        mesh=mesh,
        scratch_types=[
            pltpu.VMEM((EP // (NC * NS),), jnp.int32),
            pltpu.VMEM((NR,), jnp.float32),
        ],
        compiler_params=pltpu.CompilerParams(needs_layout_passes=False),
    )(dstA)

    dinv = pl.pallas_call(
        _dinv_body,
        out_shape=jax.ShapeDtypeStruct((NR, 1), jnp.float32),
    )(deg_parts)

    RB = 1000
    nrb = N // RB
    xg = x.reshape(G, N, F)
    Wh = jnp.stack([W[:, :HH], W[:, HH:]])
    y = pl.pallas_call(
        _xw_body,
        grid=(NC, G, nrb),
        in_specs=[
            pl.BlockSpec((1, RB, F), lambda h, g, i: (g, i, 0)),
            pl.BlockSpec((1, F, HH), lambda h, g, i: (h, 0, 0)),
            pl.BlockSpec((RB, 1), lambda h, g, i: (i, 0)),
        ],
        out_specs=pl.BlockSpec((1, 1, RB, HH), lambda h, g, i: (h, g, i, 0)),
        out_shape=jax.ShapeDtypeStruct((NC, G, N, HH), jnp.float32),
    )(xg, Wh, dinv)

    S = pl.kernel(
        functools.partial(_agg_body, ngraphs=G, nchunks=nchunks, ept=ept,
                          rows_per_tile=rows_per_tile, hh=HH, nnodes=N),
        out_type=jax.ShapeDtypeStruct((NC, G, NR, HH), jnp.float32),
        mesh=mesh,
        scratch_types=[
            pltpu.VMEM((nchunks, 128), jnp.int32),
            pltpu.VMEM((nchunks, 128), jnp.int32),
            pltpu.VMEM((128, HH), jnp.float32),
            pltpu.VMEM_SHARED((NR, HH), jnp.float32),
            pltpu.VMEM_SHARED((N, HH), jnp.float32),
            pltpu.SemaphoreType.DMA,
        ],
        compiler_params=pltpu.CompilerParams(needs_layout_passes=False,
                                            use_tc_tiling_on_sc=False),
    )(y, srcT, dstT)

    out = pl.pallas_call(
        _out_body,
        grid=(G, nrb),
        in_specs=[
            pl.BlockSpec((1, 1, RB, HH), lambda g, i: (0, g, i, 0)),
            pl.BlockSpec((1, 1, RB, HH), lambda g, i: (1, g, i, 0)),
            pl.BlockSpec((1, 1, RB, HH), lambda g, i: (0, g, i, 0)),
            pl.BlockSpec((1, 1, RB, HH), lambda g, i: (1, g, i, 0)),
            pl.BlockSpec((RB, 1), lambda g, i: (i, 0)),
            pl.BlockSpec((1, H), lambda g, i: (0, 0)),
        ],
        out_specs=pl.BlockSpec((1, RB, H), lambda g, i: (g, i, 0)),
        out_shape=jax.ShapeDtypeStruct((G, N, H), jnp.float32),
    )(S, S, y, y, dinv, b.reshape(1, H))

    return out.reshape(B, T, N, H)


# speedup vs baseline: 1.8388x; 1.0348x over previous
import functools

import jax
import jax.numpy as jnp
from jax import lax
from jax.experimental import pallas as pl
from jax.experimental.pallas import tpu as pltpu
from jax.experimental.pallas import tpu_sc as plsc

NC, NS, L = 2, 16, 16


def _deg_body(dst_hbm, degp_hbm, dstv, hist, nrows, ept):
    c = lax.axis_index("c")
    s = lax.axis_index("s")
    w = c * NS + s
    pltpu.sync_copy(dst_hbm.at[w], dstv)

    def zero(i, carry):
        hist[pl.ds(i * L, L)] = jnp.zeros((L,), jnp.float32)
        return carry

    lax.fori_loop(0, nrows // L, zero, 0)

    ones = jnp.ones((L,), jnp.float32)

    def acc(i, carry):
        idx = dstv[pl.ds(i * L, L)]
        plsc.addupdate_scatter(hist, [idx], ones)
        return carry

    lax.fori_loop(0, ept // L, acc, 0)
    pltpu.sync_copy(hist, degp_hbm.at[w])


def _dinv_body(degp_ref, dinv_ref):
    deg = jnp.sum(degp_ref[...], axis=0) + 1.0
    dinv_ref[...] = lax.rsqrt(deg)[:, None]


def _xw_body(x_ref, w_ref, dinv_ref, y_ref):
    xw = jnp.dot(x_ref[0], w_ref[0], preferred_element_type=jnp.float32)
    y_ref[0, 0] = xw * dinv_ref[...]


def _agg_body(y_hbm, src_hbm, dst_hbm, S_hbm, src_v, dst_v, b0,
              S_sh, tab_sh, g0,
              ngraphs, nchunks, ept, rows_per_tile, hh, nnodes):
    c = lax.axis_index("c")
    s = lax.axis_index("s")
    pltpu.sync_copy(src_hbm.at[s], src_v)
    pltpu.sync_copy(dst_hbm.at[s], dst_v)

    def zero(i, carry):
        r = i // (hh // L)
        l = i % (hh // L)
        b0[r, pl.ds(l * L, L)] = jnp.zeros((L,), jnp.float32)
        return carry

    base = s * rows_per_tile
    rpt = nnodes // NS
    for g in range(ngraphs):
        lax.fori_loop(0, 128 * hh // L, zero, 0)
        for k in range(rows_per_tile // 128):
            pltpu.sync_copy(b0, S_sh.at[pl.ds(base + k * 128, 128)])
        pltpu.sync_copy(y_hbm.at[c, g, pl.ds(s * rpt, rpt)],
                        tab_sh.at[pl.ds(s * rpt, rpt)])
        plsc.subcore_barrier()

        def chunk(q, carry):
            pltpu.async_copy(tab_sh.at[src_v.at[q]], b0, g0).wait()
            pltpu.sync_copy(b0, S_sh.at[dst_v.at[q]], add=True)
            return carry

        lax.fori_loop(0, nchunks, chunk, 0)
        plsc.subcore_barrier()
        for k in range(rows_per_tile // 128):
            pltpu.sync_copy(S_sh.at[pl.ds(base + k * 128, 128)],
                            S_hbm.at[c, g, pl.ds(base + k * 128, 128)])
        plsc.subcore_barrier()


def _out_body(S0_ref, S1_ref, y0_ref, y1_ref, dinv_ref, b_ref, o_ref):
    S = jnp.concatenate([S0_ref[0, 0], S1_ref[0, 0]], axis=-1)
    y = jnp.concatenate([y0_ref[0, 0], y1_ref[0, 0]], axis=-1)
    o_ref[0] = dinv_ref[...] * (S + y) + b_ref[...]


def kernel(x, edge_index, W, b):
    B, T, N, F = x.shape
    H = W.shape[1]
    G = B * T
    E = edge_index.shape[1]
    HH = H // 2

    ept = ((E // NS + 127) // 128) * 128
    nchunks = ept // 128
    EP = NS * ept
    NR = ((N + (NS * 128) - 1) // (NS * 128)) * (NS * 128)
    rows_per_tile = NR // NS

    src = edge_index[0]
    dst = edge_index[1]
    pad = EP - E
    srcp = jnp.concatenate([src, jnp.zeros((pad,), jnp.int32)])
    dstp = jnp.concatenate([dst, jnp.full((pad,), N, jnp.int32)])
    srcT = srcp.reshape(NS, nchunks, 128)
    dstT = dstp.reshape(NS, nchunks, 128)
    dstA = dstp.reshape(NC * NS, EP // (NC * NS))

    mesh = plsc.VectorSubcoreMesh(core_axis_name="c", subcore_axis_name="s")

    deg_parts = pl.kernel(
        functools.partial(_deg_body, nrows=NR, ept=EP // (NC * NS)),
        out_type=jax.ShapeDtypeStruct((NC * NS, NR), jnp.float32),
        mesh=mesh,
        scratch_types=[
            pltpu.VMEM((EP // (NC * NS),), jnp.int32),
            pltpu.VMEM((NR,), jnp.float32),
        ],
        compiler_params=pltpu.CompilerParams(needs_layout_passes=False),
    )(dstA)

    dinv = pl.pallas_call(
        _dinv_body,
        out_shape=jax.ShapeDtypeStruct((NR, 1), jnp.float32),
    )(deg_parts)

    RB = 2000
    nrb = N // RB
    xg = x.reshape(G, N, F)
    Wh = jnp.stack([W[:, :HH], W[:, HH:]])
    y = pl.pallas_call(
        _xw_body,
        grid=(NC, G, nrb),
        in_specs=[
            pl.BlockSpec((1, RB, F), lambda h, g, i: (g, i, 0)),
            pl.BlockSpec((1, F, HH), lambda h, g, i: (h, 0, 0)),
            pl.BlockSpec((RB, 1), lambda h, g, i: (i, 0)),
        ],
        out_specs=pl.BlockSpec((1, 1, RB, HH), lambda h, g, i: (h, g, i, 0)),
        out_shape=jax.ShapeDtypeStruct((NC, G, N, HH), jnp.float32),
    )(xg, Wh, dinv)

    S = pl.kernel(
        functools.partial(_agg_body, ngraphs=G, nchunks=nchunks, ept=ept,
                          rows_per_tile=rows_per_tile, hh=HH, nnodes=N),
        out_type=jax.ShapeDtypeStruct((NC, G, NR, HH), jnp.float32),
        mesh=mesh,
        scratch_types=[
            pltpu.VMEM((nchunks, 128), jnp.int32),
            pltpu.VMEM((nchunks, 128), jnp.int32),
            pltpu.VMEM((128, HH), jnp.float32),
            pltpu.VMEM_SHARED((NR, HH), jnp.float32),
            pltpu.VMEM_SHARED((N, HH), jnp.float32),
            pltpu.SemaphoreType.DMA,
        ],
        compiler_params=pltpu.CompilerParams(needs_layout_passes=False,
                                            use_tc_tiling_on_sc=False),
    )(y, srcT, dstT)

    out = pl.pallas_call(
        _out_body,
        grid=(G, nrb),
        in_specs=[
            pl.BlockSpec((1, 1, RB, HH), lambda g, i: (0, g, i, 0)),
            pl.BlockSpec((1, 1, RB, HH), lambda g, i: (1, g, i, 0)),
            pl.BlockSpec((1, 1, RB, HH), lambda g, i: (0, g, i, 0)),
            pl.BlockSpec((1, 1, RB, HH), lambda g, i: (1, g, i, 0)),
            pl.BlockSpec((RB, 1), lambda g, i: (i, 0)),
            pl.BlockSpec((1, H), lambda g, i: (0, 0)),
        ],
        out_specs=pl.BlockSpec((1, RB, H), lambda g, i: (g, i, 0)),
        out_shape=jax.ShapeDtypeStruct((G, N, H), jnp.float32),
    )(S, S, y, y, dinv, b.reshape(1, H))

    return out.reshape(B, T, N, H)
